# Initial kernel scaffold; baseline (speedup 1.0000x reference)
import jax
import jax.numpy as jnp
from jax.experimental import pallas as pl


def kernel(global_indices, table):
    raise NotImplementedError("write your pallas kernel here")


# speedup vs baseline: 1.5801x; 1.5801x over previous
import functools

import jax
import jax.numpy as jnp
from jax import lax
from jax.experimental import pallas as pl
from jax.experimental.pallas import tpu as pltpu
from jax.experimental.pallas import tpu_sc as plsc

NC = 2
NS = 16
NW = NC * NS


@functools.lru_cache(maxsize=None)
def _build(total: int, V: int, D: int, C: int):
    assert total % (NW * C) == 0
    b_per_w = total // NW
    n_chunks = b_per_w // C
    mesh = plsc.VectorSubcoreMesh(core_axis_name="c", subcore_axis_name="s")

    @functools.partial(
        pl.kernel,
        mesh=mesh,
        out_type=jax.ShapeDtypeStruct((total, D), jnp.float32),
        compiler_params=pltpu.CompilerParams(use_tc_tiling_on_sc=False),
        scratch_types=[
            pltpu.VMEM((C,), jnp.int32),
            pltpu.VMEM((C,), jnp.int32),
            pltpu.VMEM((C, D), jnp.float32),
            pltpu.VMEM((C, D), jnp.float32),
            pltpu.SemaphoreType.DMA,
            pltpu.SemaphoreType.DMA,
            pltpu.SemaphoreType.DMA,
            pltpu.SemaphoreType.DMA,
        ],
    )
    def gather_kernel(idx_hbm, table_hbm, out_hbm, idx_a, idx_b,
                      rows_a, rows_b, gsem0, gsem1, ssem0, ssem1):
        wid = lax.axis_index("s") * NC + lax.axis_index("c")
        base = wid * n_chunks
        idx_v = [idx_a, idx_b]
        rows_v = [rows_a, rows_b]
        gsem = [gsem0, gsem1]
        ssem = [ssem0, ssem1]
        gcp = [None, None]
        scp = [None, None]

        pltpu.sync_copy(idx_hbm.at[base], idx_a)
        gcp[0] = pltpu.async_copy(table_hbm.at[idx_a], rows_a, gsem[0])
        for g in range(n_chunks):
            buf = g % 2
            nbuf = (g + 1) % 2
            if g + 1 < n_chunks:
                if scp[nbuf] is not None:
                    scp[nbuf].wait()
                pltpu.sync_copy(idx_hbm.at[base + g + 1], idx_v[nbuf])
                gcp[nbuf] = pltpu.async_copy(
                    table_hbm.at[idx_v[nbuf]], rows_v[nbuf], gsem[nbuf])
            gcp[buf].wait()
            scp[buf] = pltpu.async_copy(
                rows_v[buf],
                out_hbm.at[pl.ds((wid * n_chunks + g) * C, C)],
                ssem[buf])
        for b in range(2):
            if scp[b] is not None:
                scp[b].wait()

    return gather_kernel


def kernel(global_indices, table):
    B, F = global_indices.shape
    V, D = table.shape
    total = B * F
    C = 1664
    flat = global_indices.reshape(-1).astype(jnp.int32)
    idx2 = flat.reshape(total // C, C)
    out = _build(total, V, D, C)(idx2, table)
    return out.reshape(B, F, D)
